# Initial kernel scaffold; baseline (speedup 1.0000x reference)
import jax
import jax.numpy as jnp
from jax.experimental import pallas as pl


def kernel(q, k, v, p):
    raise NotImplementedError("write your pallas kernel here")


# speedup vs baseline: 1.9729x; 1.9729x over previous
import math
import functools

import jax
import jax.numpy as jnp
import numpy as np
from jax import lax
from jax.experimental import pallas as pl
from jax.experimental.pallas import tpu as pltpu

_KERNEL_SIZE = 32
_STRIDE = 16
_SELECT_SIZE = 64
_TOP_N = 16
_NUM_INIT_BLOCKS = 1
_NUM_LOCAL_BLOCKS = 2
_KEEP = 999999.0


def _overlap_weights(n):
    num_select = math.ceil(n / _SELECT_SIZE)
    num_compress = (n - _KERNEL_SIZE) // _STRIDE + 1
    s = np.arange(num_select)
    c = np.arange(num_compress)
    select_start = s[:, None] * _SELECT_SIZE
    select_end = np.minimum(select_start + _SELECT_SIZE, n)
    compress_start = c[None, :] * _STRIDE
    compress_end = compress_start + _KERNEL_SIZE
    area = np.minimum(compress_end, select_end) - np.maximum(
        compress_start, select_start)
    return np.maximum(area, 0).astype(np.float32) / float(_STRIDE)


def _topk_body(p_ref, wt_ref, idx_ref, *, num_select, topn):
    sp = jnp.dot(p_ref[...], wt_ref[...], preferred_element_type=jnp.float32)
    rows = sp.shape[0]
    iota = lax.broadcasted_iota(jnp.int32, (rows, num_select), 1)
    forced = (iota < _NUM_INIT_BLOCKS) | (iota >= num_select - _NUM_LOCAL_BLOCKS)
    sp = jnp.where(forced, _KEEP, sp)
    cols = []
    for _ in range(topn):
        mx = jnp.max(sp, axis=1, keepdims=True)
        cand = jnp.where(sp == mx, iota, num_select)
        sel = jnp.min(cand, axis=1, keepdims=True)
        cols.append(sel)
        sp = jnp.where(iota == sel, -jnp.inf, sp)
    idx_ref[...] = jnp.concatenate(cols, axis=1)


def _attn_body(idx_ref, q_ref, k_hbm, v_hbm, o_ref, kbuf, vbuf, ksem, vsem,
               *, qh, topn, blk, scale):
    g = pl.program_id(0)
    bb = g // qh
    hh = g % qh
    copies = []
    for j in range(topn):
        off = idx_ref[g, j] * blk
        ck = pltpu.make_async_copy(
            k_hbm.at[bb, pl.ds(off, blk), hh],
            kbuf.at[pl.ds(j * blk, blk), :], ksem)
        cv = pltpu.make_async_copy(
            v_hbm.at[bb, pl.ds(off, blk), hh],
            vbuf.at[pl.ds(j * blk, blk), :], vsem)
        ck.start()
        cv.start()
        copies.append((ck, cv))
    for ck, cv in copies:
        ck.wait()
        cv.wait()
    qv = q_ref[0]
    s = lax.dot_general(kbuf[...], qv, (((1,), (1,)), ((), ())),
                        preferred_element_type=jnp.float32) * scale
    mx = jnp.max(s)
    e = jnp.exp(s - mx)
    denom = jnp.sum(e)
    o = lax.dot_general(e, vbuf[...], (((0,), (0,)), ((), ())),
                        preferred_element_type=jnp.float32)
    o_ref[0] = o / denom


def kernel(q, k, v, p):
    b, m, qh, d = q.shape
    _, n, kh, _ = k.shape
    num_select = math.ceil(n / _SELECT_SIZE)
    num_compress = (n - _KERNEL_SIZE) // _STRIDE + 1
    g_total = b * qh
    kc_pad = ((num_compress + 127) // 128) * 128

    p_r = p.reshape(g_total, num_compress)
    p_pad = jnp.pad(p_r, ((0, 0), (0, kc_pad - num_compress)))
    wt = jnp.asarray(
        np.pad(_overlap_weights(n).T, ((0, kc_pad - num_compress), (0, 0))))

    idx = pl.pallas_call(
        functools.partial(_topk_body, num_select=num_select, topn=_TOP_N),
        out_shape=jax.ShapeDtypeStruct((g_total, _TOP_N), jnp.int32),
    )(p_pad, wt)

    scale = d ** (-0.5)
    q3 = q.reshape(g_total, 1, d)
    grid_spec = pltpu.PrefetchScalarGridSpec(
        num_scalar_prefetch=1,
        grid=(g_total,),
        in_specs=[
            pl.BlockSpec((1, 1, d), lambda g, idx_s: (g, 0, 0)),
            pl.BlockSpec(memory_space=pl.ANY),
            pl.BlockSpec(memory_space=pl.ANY),
        ],
        out_specs=pl.BlockSpec((1, 1, d), lambda g, idx_s: (g, 0, 0)),
        scratch_shapes=[
            pltpu.VMEM((_TOP_N * _SELECT_SIZE, d), jnp.float32),
            pltpu.VMEM((_TOP_N * _SELECT_SIZE, d), jnp.float32),
            pltpu.SemaphoreType.DMA,
            pltpu.SemaphoreType.DMA,
        ],
    )
    out = pl.pallas_call(
        functools.partial(_attn_body, qh=qh, topn=_TOP_N, blk=_SELECT_SIZE,
                          scale=scale),
        grid_spec=grid_spec,
        out_shape=jax.ShapeDtypeStruct((g_total, 1, d), jnp.float32),
    )(idx, q3, k, v)
    return out.reshape(b, m, qh, d)


# speedup vs baseline: 3.7501x; 1.9008x over previous
import math
import functools

import jax
import jax.numpy as jnp
import numpy as np
from jax import lax
from jax.experimental import pallas as pl
from jax.experimental.pallas import tpu as pltpu

_KERNEL_SIZE = 32
_STRIDE = 16
_SELECT_SIZE = 64
_TOP_N = 16
_NUM_INIT_BLOCKS = 1
_NUM_LOCAL_BLOCKS = 2
_KEEP = 999999.0


def _overlap_weights(n):
    num_select = math.ceil(n / _SELECT_SIZE)
    num_compress = (n - _KERNEL_SIZE) // _STRIDE + 1
    s = np.arange(num_select)
    c = np.arange(num_compress)
    select_start = s[:, None] * _SELECT_SIZE
    select_end = np.minimum(select_start + _SELECT_SIZE, n)
    compress_start = c[None, :] * _STRIDE
    compress_end = compress_start + _KERNEL_SIZE
    area = np.minimum(compress_end, select_end) - np.maximum(
        compress_start, select_start)
    return np.maximum(area, 0).astype(np.float32) / float(_STRIDE)


def _topk_body(p_ref, wt_ref, idx_ref, *, num_select, topn):
    sp = jnp.dot(p_ref[...], wt_ref[...], preferred_element_type=jnp.float32)
    rows = sp.shape[0]
    iota = lax.broadcasted_iota(jnp.int32, (rows, num_select), 1)
    forced = (iota < _NUM_INIT_BLOCKS) | (iota >= num_select - _NUM_LOCAL_BLOCKS)
    sp = jnp.where(forced, _KEEP, sp)
    cols = []
    for _ in range(topn):
        mx = jnp.max(sp, axis=1, keepdims=True)
        cand = jnp.where(sp == mx, iota, num_select)
        sel = jnp.min(cand, axis=1, keepdims=True)
        cols.append(sel)
        sp = jnp.where(iota == sel, -jnp.inf, sp)
    idx_ref[...] = jnp.concatenate(cols, axis=1)


def _attn_body(idx_ref, q_ref, k_hbm, v_hbm, o_ref, kbuf, vbuf, ksem, vsem,
               *, qh, topn, blk, scale):
    g = pl.program_id(0)
    ng = pl.num_programs(0)
    span = topn * blk

    def fire(step, slot):
        bb = step // qh
        hh = step % qh
        for j in range(topn):
            off = idx_ref[step, j] * blk
            pltpu.make_async_copy(
                k_hbm.at[bb, pl.ds(off, blk), hh],
                kbuf.at[slot, pl.ds(j * blk, blk), :], ksem.at[slot]).start()
            pltpu.make_async_copy(
                v_hbm.at[bb, pl.ds(off, blk), hh],
                vbuf.at[slot, pl.ds(j * blk, blk), :], vsem.at[slot]).start()

    slot = lax.rem(g, 2)
    nslot = lax.rem(g + 1, 2)

    @pl.when(g == 0)
    def _():
        fire(g, slot)

    @pl.when(g + 1 < ng)
    def _():
        fire(g + 1, nslot)

    pltpu.make_async_copy(
        k_hbm.at[0, pl.ds(0, span), 0], kbuf.at[slot], ksem.at[slot]).wait()
    pltpu.make_async_copy(
        v_hbm.at[0, pl.ds(0, span), 0], vbuf.at[slot], vsem.at[slot]).wait()

    qv = q_ref[0]
    s = lax.dot_general(qv, kbuf[slot], (((1,), (1,)), ((), ())),
                        preferred_element_type=jnp.float32) * scale
    mx = jnp.max(s)
    e = jnp.exp(s - mx)
    denom = jnp.sum(e)
    o = lax.dot_general(e, vbuf[slot], (((1,), (0,)), ((), ())),
                        preferred_element_type=jnp.float32)
    o_ref[0] = o / denom


def kernel(q, k, v, p):
    b, m, qh, d = q.shape
    _, n, kh, _ = k.shape
    num_select = math.ceil(n / _SELECT_SIZE)
    num_compress = (n - _KERNEL_SIZE) // _STRIDE + 1
    g_total = b * qh
    kc_pad = ((num_compress + 127) // 128) * 128

    p_r = p.reshape(g_total, num_compress)
    p_pad = jnp.pad(p_r, ((0, 0), (0, kc_pad - num_compress)))
    wt = jnp.asarray(
        np.pad(_overlap_weights(n).T, ((0, kc_pad - num_compress), (0, 0))))

    idx = pl.pallas_call(
        functools.partial(_topk_body, num_select=num_select, topn=_TOP_N),
        out_shape=jax.ShapeDtypeStruct((g_total, _TOP_N), jnp.int32),
    )(p_pad, wt)

    scale = d ** (-0.5)
    q3 = q.reshape(g_total, 1, d)
    grid_spec = pltpu.PrefetchScalarGridSpec(
        num_scalar_prefetch=1,
        grid=(g_total,),
        in_specs=[
            pl.BlockSpec((1, 1, d), lambda g, idx_s: (g, 0, 0)),
            pl.BlockSpec(memory_space=pl.ANY),
            pl.BlockSpec(memory_space=pl.ANY),
        ],
        out_specs=pl.BlockSpec((1, 1, d), lambda g, idx_s: (g, 0, 0)),
        scratch_shapes=[
            pltpu.VMEM((2, _TOP_N * _SELECT_SIZE, d), jnp.float32),
            pltpu.VMEM((2, _TOP_N * _SELECT_SIZE, d), jnp.float32),
            pltpu.SemaphoreType.DMA((2,)),
            pltpu.SemaphoreType.DMA((2,)),
        ],
    )
    out = pl.pallas_call(
        functools.partial(_attn_body, qh=qh, topn=_TOP_N, blk=_SELECT_SIZE,
                          scale=scale),
        grid_spec=grid_spec,
        out_shape=jax.ShapeDtypeStruct((g_total, 1, d), jnp.float32),
    )(idx, q3, k, v)
    return out.reshape(b, m, qh, d)


# speedup vs baseline: 5.2419x; 1.3978x over previous
import math
import functools

import jax
import jax.numpy as jnp
import numpy as np
from jax import lax
from jax.experimental import pallas as pl
from jax.experimental.pallas import tpu as pltpu

_KERNEL_SIZE = 32
_STRIDE = 16
_SELECT_SIZE = 64
_TOP_N = 16
_NUM_INIT_BLOCKS = 1
_NUM_LOCAL_BLOCKS = 2
_KEEP = 999999.0


def _overlap_weights(n):
    num_select = math.ceil(n / _SELECT_SIZE)
    num_compress = (n - _KERNEL_SIZE) // _STRIDE + 1
    s = np.arange(num_select)
    c = np.arange(num_compress)
    select_start = s[:, None] * _SELECT_SIZE
    select_end = np.minimum(select_start + _SELECT_SIZE, n)
    compress_start = c[None, :] * _STRIDE
    compress_end = compress_start + _KERNEL_SIZE
    area = np.minimum(compress_end, select_end) - np.maximum(
        compress_start, select_start)
    return np.maximum(area, 0).astype(np.float32) / float(_STRIDE)


def _topk_body(p_ref, wt_ref, idx_ref, *, num_select, topn):
    sp = jnp.dot(p_ref[...], wt_ref[...], preferred_element_type=jnp.float32)
    rows = sp.shape[0]
    iota = lax.broadcasted_iota(jnp.int32, (rows, num_select), 1)
    forced = (iota < _NUM_INIT_BLOCKS) | (iota >= num_select - _NUM_LOCAL_BLOCKS)
    sp = jnp.where(forced, _KEEP, sp)
    cols = []
    for _ in range(topn):
        mx = jnp.max(sp, axis=1, keepdims=True)
        cand = jnp.where(sp == mx, iota, num_select)
        sel = jnp.min(cand, axis=1, keepdims=True)
        cols.append(sel)
        sp = jnp.where(iota == sel, -jnp.inf, sp)
    idx_ref[...] = jnp.concatenate(cols, axis=1)


def _attn_body(idx_ref, q_ref, k_hbm, v_hbm, o_ref, kbuf, vbuf, ksem, vsem,
               *, qh, topn, blk, scale):
    g = pl.program_id(0)
    ng = pl.num_programs(0)
    span = topn * blk

    def fire(step, slot):
        bb = step // qh
        hh = step % qh
        for j in range(topn):
            off = idx_ref[step, j] * blk
            pltpu.make_async_copy(
                k_hbm.at[bb, pl.ds(off, blk), hh],
                kbuf.at[slot, pl.ds(j * blk, blk), :], ksem.at[slot]).start()
            pltpu.make_async_copy(
                v_hbm.at[bb, pl.ds(off, blk), hh],
                vbuf.at[slot, pl.ds(j * blk, blk), :], vsem.at[slot]).start()

    nbuf = 4
    slot = lax.rem(g, nbuf)

    @pl.when(g == 0)
    def _():
        for s in range(nbuf - 1):
            fire(s, s)

    @pl.when(g + nbuf - 1 < ng)
    def _():
        fire(g + nbuf - 1, lax.rem(g + nbuf - 1, nbuf))

    pltpu.make_async_copy(
        k_hbm.at[0, pl.ds(0, span), 0], kbuf.at[slot], ksem.at[slot]).wait()
    pltpu.make_async_copy(
        v_hbm.at[0, pl.ds(0, span), 0], vbuf.at[slot], vsem.at[slot]).wait()

    qv = q_ref[0]
    s = lax.dot_general(qv, kbuf[slot], (((1,), (1,)), ((), ())),
                        preferred_element_type=jnp.float32) * scale
    mx = jnp.max(s)
    e = jnp.exp(s - mx)
    denom = jnp.sum(e)
    o = lax.dot_general(e, vbuf[slot], (((1,), (0,)), ((), ())),
                        preferred_element_type=jnp.float32)
    o_ref[0] = o / denom


def kernel(q, k, v, p):
    b, m, qh, d = q.shape
    _, n, kh, _ = k.shape
    num_select = math.ceil(n / _SELECT_SIZE)
    num_compress = (n - _KERNEL_SIZE) // _STRIDE + 1
    g_total = b * qh
    kc_pad = ((num_compress + 127) // 128) * 128

    p_r = p.reshape(g_total, num_compress)
    p_pad = jnp.pad(p_r, ((0, 0), (0, kc_pad - num_compress)))
    wt = jnp.asarray(
        np.pad(_overlap_weights(n).T, ((0, kc_pad - num_compress), (0, 0))))

    idx = pl.pallas_call(
        functools.partial(_topk_body, num_select=num_select, topn=_TOP_N),
        out_shape=jax.ShapeDtypeStruct((g_total, _TOP_N), jnp.int32),
    )(p_pad, wt)

    scale = d ** (-0.5)
    q3 = q.reshape(g_total, 1, d)
    grid_spec = pltpu.PrefetchScalarGridSpec(
        num_scalar_prefetch=1,
        grid=(g_total,),
        in_specs=[
            pl.BlockSpec((1, 1, d), lambda g, idx_s: (g, 0, 0)),
            pl.BlockSpec(memory_space=pl.ANY),
            pl.BlockSpec(memory_space=pl.ANY),
        ],
        out_specs=pl.BlockSpec((1, 1, d), lambda g, idx_s: (g, 0, 0)),
        scratch_shapes=[
            pltpu.VMEM((4, _TOP_N * _SELECT_SIZE, d), jnp.float32),
            pltpu.VMEM((4, _TOP_N * _SELECT_SIZE, d), jnp.float32),
            pltpu.SemaphoreType.DMA((4,)),
            pltpu.SemaphoreType.DMA((4,)),
        ],
    )
    out = pl.pallas_call(
        functools.partial(_attn_body, qh=qh, topn=_TOP_N, blk=_SELECT_SIZE,
                          scale=scale),
        grid_spec=grid_spec,
        out_shape=jax.ShapeDtypeStruct((g_total, 1, d), jnp.float32),
    )(idx, q3, k, v)
    return out.reshape(b, m, qh, d)
